# baseline (speedup 1.0000x reference)
import functools

import jax
import jax.numpy as jnp
from jax import lax
from jax.experimental import pallas as pl
from jax.experimental.pallas import tpu as pltpu
from jax.experimental.pallas import tpu_sc as plsc

_NU = 100000
_L = 16
_NF = 200000
_D = 64
_NCOLS = (_NF + 127) // 128
_PHIST = 2064
_SCAP = 1168
_TRASH = 8192


@functools.cache
def _build_fm_kernel(batch: int):
  info = plsc.get_sparse_core_info()
  nc, ns = info.num_cores, info.num_subcores
  bpw = batch // (nc * ns)
  half = batch // nc
  npairs = 2 * bpw
  nvec = npairs // _L

  mesh = plsc.VectorSubcoreMesh(core_axis_name="c", subcore_axis_name="s")

  @functools.partial(
      pl.kernel,
      out_type=jax.ShapeDtypeStruct((batch,), jnp.float32),
      mesh=mesh,
      compiler_params=pltpu.CompilerParams(
          needs_layout_passes=False, use_tc_tiling_on_sc=True),
      scratch_types=[
          pltpu.VMEM((bpw,), jnp.int32),
          pltpu.VMEM((bpw,), jnp.int32),
          pltpu.VMEM((npairs,), jnp.int32),
          pltpu.VMEM((_SCAP,), jnp.int32),
          pltpu.VMEM((_PHIST,), jnp.int32),
          pltpu.VMEM((_PHIST,), jnp.int32),
          pltpu.VMEM((_PHIST,), jnp.int32),
          pltpu.VMEM((_L,), jnp.int32),
          pltpu.VMEM((16 * 136 + 16,), jnp.int32),
          pltpu.VMEM((32,), jnp.int32),
          pltpu.VMEM((16 * _SCAP + 16,), jnp.int32),
          pltpu.VMEM((128, 128), jnp.float32),
          pltpu.VMEM((64, 128), jnp.float32),
          pltpu.VMEM((64,), jnp.int32),
          pltpu.VMEM((bpw,), jnp.float32),
          pltpu.VMEM((bpw,), jnp.float32),
          pltpu.VMEM((_L,), jnp.float32),
          pltpu.VMEM((_L * _L,), jnp.float32),
          pltpu.VMEM((bpw,), jnp.float32),
          pltpu.VMEM_SHARED((16 * _SCAP,), jnp.int32),
          pltpu.VMEM_SHARED((16 * _PHIST,), jnp.int32),
          pltpu.VMEM_SHARED((half + 256, 128), jnp.float32),
          pltpu.SemaphoreType.DMA,
          pltpu.SemaphoreType.DMA,
          pltpu.SemaphoreType.DMA,
      ],
  )
  def fm(uid_hbm, iid_hbm, bias_hbm, lw_hbm, embt_hbm, out_hbm,
         uidx, iidx, pairs, sortv, hist, offs, cnt, tmp16, owin, hbase,
         hits, slab, stage, slotb, lwu, lwi, bias_v, sums, out_v,
         mbox, offs_tbl, results, sem_lw, sem_a, sem_b):
    cid = lax.axis_index("c")
    sid = lax.axis_index("s")
    gbase = (cid * ns + sid) * bpw
    lane = lax.iota(jnp.int32, _L)

    def sread(ref, idx):
      return plsc.load_gather(ref, [jnp.broadcast_to(idx, (_L,))])[0]

    def al8(x):
      return pl.multiple_of(x, 8)

    pltpu.sync_copy(uid_hbm.at[pl.ds(al8(gbase), bpw)], uidx)
    pltpu.sync_copy(iid_hbm.at[pl.ds(al8(gbase), bpw)], iidx)
    pltpu.sync_copy(bias_hbm, bias_v)
    for t in range(bpw // _L):
      sl = pl.ds(t * _L, _L)
      iidx[sl] = iidx[sl] + _NU
    lw_copies = []
    for c in range(bpw // 128):
      sl = pl.ds(c * 128, 128)
      lw_copies.append(
          pltpu.async_copy(lw_hbm.at[uidx.at[sl]], lwu.at[sl], sem_lw))
      lw_copies.append(
          pltpu.async_copy(lw_hbm.at[iidx.at[sl]], lwi.at[sl], sem_lw))

    lb0 = sid * bpw

    def pack(t, _):
      sl = pl.ds(al8(t * _L), _L)
      lb = lb0 + t * _L + lane
      pairs[sl] = (uidx[sl] << 13) | lb
      pairs[pl.ds(al8(bpw + t * _L), _L)] = (iidx[sl] << 13) | lb
      return _

    lax.fori_loop(0, bpw // _L, pack, None)

    def zero_hist(t, _):
      hist[pl.ds(al8(t * _L), _L)] = jnp.zeros((_L,), jnp.int32)
      return _

    lax.fori_loop(0, _PHIST // _L, zero_hist, None)

    def rank_info(k):
      tmp16[pl.ds(0, _L)] = k
      prev = plsc.load_gather(tmp16, [jnp.maximum(lane - 1, 0)])
      nxt = plsc.load_gather(tmp16, [jnp.minimum(lane + 1, _L - 1)])
      is_new = jnp.logical_or(k != prev, lane == 0)
      is_last = jnp.logical_or(k != nxt, lane == _L - 1)
      firsts = plsc.cummax(jnp.where(is_new, lane, 0))
      rank = lane - firsts
      return rank, is_last

    def hist_pass(t, _):
      v = pairs[pl.ds(al8(t * _L), _L)]
      k, v = plsc.sort_key_val(v >> 20, v)
      rank, is_last = rank_info(k)
      plsc.addupdate_scatter(hist, [k], rank + 1, mask=is_last)
      return _

    lax.fori_loop(0, nvec, hist_pass, None)

    def excl_scan(t, carry):
      sl = pl.ds(al8(t * _L), _L)
      h = hist[sl]
      incl = plsc.cumsum(h)
      offs[sl] = incl - h + carry
      cnt[sl] = incl - h + carry
      return carry + incl[_L - 1]

    lax.fori_loop(0, _PHIST // _L, excl_scan, jnp.int32(0))

    def place_pass(t, _):
      v = pairs[pl.ds(al8(t * _L), _L)]
      k, v = plsc.sort_key_val(v >> 20, v)
      rank, is_last = rank_info(k)
      pos = plsc.load_gather(cnt, [k]) + rank
      plsc.store_scatter(sortv, [pos], v)
      plsc.addupdate_scatter(cnt, [k], rank + 1, mask=is_last)
      return _

    lax.fori_loop(0, nvec, place_pass, None)

    pltpu.sync_copy(offs, offs_tbl.at[pl.ds(al8(sid * _PHIST), _PHIST)])
    pltpu.sync_copy(sortv, mbox.at[pl.ds(al8(sid * _SCAP), _SCAP)])

    def zero_stage(r, _):
      for t in range(8):
        stage[r, pl.ds(t * _L, _L)] = jnp.zeros((_L,), jnp.float32)
      return _

    lax.fori_loop(0, 64, zero_stage, None)

    for j in range(8):
      pltpu.sync_copy(stage, results.at[pl.ds(al8(sid * 528 + j * 64), 64), :])
    pltpu.sync_copy(stage.at[pl.ds(0, _L), :],
                    results.at[pl.ds(al8(sid * 528 + 512), _L), :])

    plsc.subcore_barrier()

    @pl.when(sid <= (_NCOLS - 1) // 128)
    def _owner_work():
      def recv_win(s, _):
        pltpu.sync_copy(
            offs_tbl.at[pl.ds(al8(s * _PHIST + sid * 128), 136)],
            owin.at[pl.ds(al8(s * 136), 136)])
        return _

      lax.fori_loop(0, 16, recv_win, None)

      def recv_seg(s, run):
        e0 = sread(owin, s * 136)
        e128 = sread(owin, s * 136 + 128)
        a0 = e0 & ~7
        units = (((e128 + 7) & ~7) - a0 + 127) // 128

        def recv_unit(u, _u):
          pltpu.sync_copy(
              mbox.at[pl.ds(al8(s * _SCAP + a0 + u * 128), 128)],
              hits.at[pl.ds(al8(run + u * 128), 128)])
          return _u

        lax.fori_loop(0, units, recv_unit, None)
        hb = hbase[pl.ds(0, _L)]
        hbase[pl.ds(0, _L)] = jnp.where(lane == s, run - a0, hb)
        return run + units * 128

      lax.fori_loop(0, 16, recv_seg, jnp.int32(0))

      for t in range(4):
        slotb[pl.ds(t * _L, _L)] = jnp.full((_L,), _TRASH, jnp.int32)

      col_hi = _NCOLS - 1

      col0 = jnp.minimum(sid * 128, col_hi)
      pltpu.async_copy(
          embt_hbm.at[pl.ds(0, _D),
                      pl.ds(pl.multiple_of(col0 * 128, 128), 128)],
          slab.at[pl.ds(0, _D), :], sem_a)

      def piece_body(p, hh):
        nbase = ((p + 1) & 1) * _D

        @pl.when(jnp.logical_and(p < 127, ((p + 1) & 1) == 0))
        def _():
          col = jnp.minimum(sid * 128 + p + 1, col_hi)
          pltpu.async_copy(
              embt_hbm.at[pl.ds(0, _D),
                          pl.ds(pl.multiple_of(col * 128, 128), 128)],
              slab.at[pl.ds(nbase, _D), :], sem_a)

        @pl.when(jnp.logical_and(p < 127, ((p + 1) & 1) == 1))
        def _():
          col = jnp.minimum(sid * 128 + p + 1, col_hi)
          pltpu.async_copy(
              embt_hbm.at[pl.ds(0, _D),
                          pl.ds(pl.multiple_of(col * 128, 128), 128)],
              slab.at[pl.ds(nbase, _D), :], sem_b)

        @pl.when((p & 1) == 0)
        def _():
          pltpu.make_async_copy(
              embt_hbm.at[pl.ds(0, _D), pl.ds(0, 128)],
              slab.at[pl.ds(0, _D), :], sem_a).wait()

        @pl.when((p & 1) == 1)
        def _():
          pltpu.make_async_copy(
              embt_hbm.at[pl.ds(0, _D), pl.ds(0, 128)],
              slab.at[pl.ds(_D, _D), :], sem_b).wait()

        base = (p & 1) * _D

        def sender_hits(s, hh_s):
          hb = sread(hbase, s)
          st = sread(owin, s * 136 + p) + hb
          en = sread(owin, s * 136 + p + 1) + hb

          def one_hit(h, hh_h):
            v = sread(hits, h)
            slot = v & 8191
            ln = (v >> 13) & 127
            hh64 = hh_h & 63
            q = None
            for k in range(_D // _L):
              vals = plsc.load_gather(
                  slab, [base + k * _L + lane, jnp.broadcast_to(ln, (_L,))])
              stage[hh64, pl.ds(k * _L, _L)] = vals
              q = vals * vals if q is None else q + vals * vals
            stage[hh64, pl.ds(_D, _L)] = q
            sb_base = al8(hh64 & ~(_L - 1))
            sb = slotb[pl.ds(sb_base, _L)]
            slotb[pl.ds(sb_base, _L)] = jnp.where(
                lane == (hh64 & (_L - 1)), slot, sb)

            @pl.when(hh64 == 63)
            def _():
              pltpu.sync_copy(stage, results.at[slotb], add=True)
              for t in range(4):
                slotb[pl.ds(t * _L, _L)] = jnp.full((_L,), _TRASH, jnp.int32)

            return hh_h + 1

          return lax.fori_loop(st, en, one_hit, hh_s)

        return lax.fori_loop(0, 16, sender_hits, hh)

      hh_end = lax.fori_loop(0, 128, piece_body, jnp.int32(0))

      @pl.when((hh_end & 63) != 0)
      def _():
        pltpu.sync_copy(stage, results.at[slotb], add=True)

    plsc.subcore_barrier()

    for cp in lw_copies:
      cp.wait()
    bias_vec = bias_v[pl.ds(0, _L)]
    row_base = lane * _L

    def chunk_body(ch, _):
      pltpu.sync_copy(
          results.at[pl.ds(al8(sid * bpw + ch * 64), 64), :], stage)

      def group_body(g, _g):
        for j in range(_L):
          m = g * _L + j
          s = None
          for k in range(_D // _L):
            v = stage[m, pl.ds(k * _L, _L)]
            s = v * v if s is None else s + v * v
          sums[pl.ds(j * _L, _L)] = s - stage[m, pl.ds(_D, _L)]
        osl = pl.ds(al8(ch * 64 + g * _L), _L)
        acc = None
        for c in range(_L):
          col = plsc.load_gather(sums, [row_base + c])
          acc = col if acc is None else acc + col
        out_v[osl] = bias_vec + lwu[osl] + lwi[osl] + 0.5 * acc
        return _g

      lax.fori_loop(0, 4, group_body, None)
      return _

    lax.fori_loop(0, bpw // 64, chunk_body, None)
    pltpu.sync_copy(out_v, out_hbm.at[pl.ds(al8(gbase), bpw)])

  return fm


def kernel(user_ids, item_ids, global_bias, linear_weights, embeddings):
  batch = user_ids.shape[0]
  num_features = embeddings.shape[0]
  lw_flat = linear_weights.reshape(num_features)
  bias16 = jnp.broadcast_to(global_bias.astype(jnp.float32), (_L,))
  fm = _build_fm_kernel(batch)
  out = fm(user_ids.astype(jnp.int32), item_ids.astype(jnp.int32),
           bias16, lw_flat, embeddings.T)
  return out.reshape(batch, 1)


# speedup vs baseline: 1.5048x; 1.5048x over previous
import functools

import jax
import jax.numpy as jnp
from jax import lax
from jax.experimental import pallas as pl
from jax.experimental.pallas import tpu as pltpu
from jax.experimental.pallas import tpu_sc as plsc

_NUM_USERS = 100000
_LANES = 16
_IDX_CHUNK = 128


@functools.cache
def _build_fm_kernel(batch: int, num_features: int, embed_dim: int):
  info = plsc.get_sparse_core_info()
  num_workers = info.num_cores * info.num_subcores
  bpw = batch // num_workers
  assert batch % (8 * num_workers) == 0
  assert embed_dim % _LANES == 0
  n_chunks = bpw // _IDX_CHUNK
  n_groups = bpw // _LANES
  d_vecs = embed_dim // _LANES

  mesh = plsc.VectorSubcoreMesh(core_axis_name="c", subcore_axis_name="s")

  @functools.partial(
      pl.kernel,
      out_type=jax.ShapeDtypeStruct((batch,), jnp.float32),
      mesh=mesh,
      compiler_params=pltpu.CompilerParams(
          needs_layout_passes=False, use_tc_tiling_on_sc=False),
      scratch_types=[
          pltpu.VMEM((bpw,), jnp.int32),
          pltpu.VMEM((bpw,), jnp.int32),
          pltpu.VMEM((bpw, embed_dim), jnp.float32),
          pltpu.VMEM((bpw, embed_dim), jnp.float32),
          pltpu.VMEM((bpw,), jnp.float32),
          pltpu.VMEM((bpw,), jnp.float32),
          pltpu.VMEM((_LANES,), jnp.float32),
          pltpu.VMEM((_LANES * _LANES,), jnp.float32),
          pltpu.VMEM((bpw,), jnp.float32),
          pltpu.SemaphoreType.DMA,
      ],
  )
  def fm(uid_hbm, iid_hbm, bias_hbm, lw_hbm, emb_hbm, out_hbm,
         uidx_v, iidx_v, rows_u, rows_i, lwu_v, lwi_v, bias_v, sums_v,
         out_v, sem):
    wid = lax.axis_index("s") * info.num_cores + lax.axis_index("c")
    base = wid * bpw

    pltpu.sync_copy(uid_hbm.at[pl.ds(base, bpw)], uidx_v)
    pltpu.sync_copy(iid_hbm.at[pl.ds(base, bpw)], iidx_v)
    pltpu.sync_copy(bias_hbm, bias_v)

    for k in range(bpw // _LANES):
      sl = pl.ds(k * _LANES, _LANES)
      iidx_v[sl] = iidx_v[sl] + _NUM_USERS

    copies = []
    for c in range(n_chunks):
      sl = pl.ds(c * _IDX_CHUNK, _IDX_CHUNK)
      copies.append(
          pltpu.async_copy(emb_hbm.at[uidx_v.at[sl]], rows_u.at[sl], sem))
      copies.append(
          pltpu.async_copy(emb_hbm.at[iidx_v.at[sl]], rows_i.at[sl], sem))
      copies.append(
          pltpu.async_copy(lw_hbm.at[uidx_v.at[sl]], lwu_v.at[sl], sem))
      copies.append(
          pltpu.async_copy(lw_hbm.at[iidx_v.at[sl]], lwi_v.at[sl], sem))
    for cp in copies:
      cp.wait()

    lane = lax.iota(jnp.int32, _LANES)
    bias_vec = bias_v[pl.ds(0, _LANES)]

    def group_body(g, _):
      for j in range(_LANES):
        r = g * _LANES + j
        s = None
        for k in range(d_vecs):
          dsl = pl.ds(k * _LANES, _LANES)
          p = rows_u[r, dsl] * rows_i[r, dsl]
          s = p if s is None else s + p
        sums_v[pl.ds(j * _LANES, _LANES)] = s
      gsl = pl.ds(g * _LANES, _LANES)
      acc = bias_vec + lwu_v[gsl] + lwi_v[gsl]
      row_base = lane * _LANES
      for c in range(_LANES):
        acc = acc + plsc.load_gather(sums_v, [row_base + c])
      out_v[gsl] = acc
      return _

    lax.fori_loop(0, n_groups, group_body, None)
    pltpu.sync_copy(out_v, out_hbm.at[pl.ds(base, bpw)])

  return fm


def kernel(user_ids, item_ids, global_bias, linear_weights, embeddings):
  batch = user_ids.shape[0]
  num_features, embed_dim = embeddings.shape
  lw_flat = linear_weights.reshape(num_features)
  bias16 = jnp.broadcast_to(global_bias.astype(jnp.float32), (_LANES,))
  fm = _build_fm_kernel(batch, num_features, embed_dim)
  out = fm(user_ids.astype(jnp.int32), item_ids.astype(jnp.int32),
           bias16, lw_flat, embeddings)
  return out.reshape(batch, 1)
